# Initial kernel scaffold; baseline (speedup 1.0000x reference)
import jax
import jax.numpy as jnp
from jax.experimental import pallas as pl


def kernel(tensor, W1, att_src1, att_dst1, W_edge1, att_edge1, b1, W2, att_src2, att_dst2, W_edge2, att_edge2, b2, W3, att_src3, att_dst3, W_edge3, att_edge3, b3, fc1_w, fc1_b, fc2_w, fc2_b):
    raise NotImplementedError("write your pallas kernel here")


# speedup vs baseline: 69.2147x; 69.2147x over previous
import functools

import jax
import jax.numpy as jnp
from jax.experimental import pallas as pl

B = 1024
NUM_AGENTS = 4
MAX_OBS = 60
NNF = 8
HID = 16
HEADS = 8
OUT_CH = 2
NPG = NUM_AGENTS + MAX_OBS
HF = HEADS * HID
NFL = NPG * NNF

GB = 32


def _bmm(a, b):
    return jax.lax.dot_general(
        a, b, (((2,), (1,)), ((0,), (0,))), preferred_element_type=jnp.float32, precision=jax.lax.Precision.HIGHEST)


def _scat(oh, v):
    return jax.lax.dot_general(
        v, oh, (((1,), (1,)), ((0,), (0,))), preferred_element_type=jnp.float32,
        precision=jax.lax.Precision.HIGHEST).transpose(0, 2, 1)


def _mm(a, b, prec=jax.lax.Precision.HIGHEST):
    return jax.lax.dot_general(
        a, b, (((a.ndim - 1,), (0,)), ((), ())), preferred_element_type=jnp.float32, precision=prec)


def _leaky(a):
    return jnp.where(a > 0, a, 0.2 * a)


def _gat_body(x0_ref, srcf_ref, dstf_ref, ea_ref,
              W1_ref, As1_ref, Ad1_ref, c1_ref, b1_ref,
              W2_ref, As2_ref, Ad2_ref, c2_ref, b2_ref,
              W3_ref, As3_ref, Ad3_ref, c3_ref, b3_ref,
              R_ref, fc1w_ref, fc1b_ref, fc2w_ref, fc2b_ref,
              o_ref):
    f32 = jnp.float32
    srcf = srcf_ref[...]
    dstf = dstf_ref[...]
    ea = ea_ref[...]
    R = R_ref[...]

    iota = jax.lax.broadcasted_iota(jnp.int32, (GB, MAX_OBS, NPG), 2)
    SrcOH = (srcf.astype(jnp.int32)[:, :, None] == iota).astype(f32)
    DstOH = (dstf.astype(jnp.int32)[:, :, None] == iota).astype(f32)

    ea_sum = jax.lax.dot_general(
        ea, DstOH, (((1,), (1,)), ((0,), (0,))),
        preferred_element_type=f32,
        precision=jax.lax.Precision.HIGHEST)
    cnt = jnp.sum(DstOH, axis=1)
    loop_attr = ea_sum / jnp.maximum(cnt, 1.0)

    def gat(x, W_r, As_r, Ad_r, c_r, b_r, add_loops):
        xp = _mm(x, W_r[...], jax.lax.Precision.DEFAULT)
        s_src = _mm(xp, As_r[...])
        s_dst = _mm(xp, Ad_r[...])
        c = c_r[...]
        alpha = _bmm(SrcOH, s_src) + _bmm(DstOH, s_dst) + ea[:, :, None] * c
        e = jnp.exp(_leaky(alpha))
        denom = _scat(DstOH, e)
        if add_loops:
            e_l = jnp.exp(_leaky(s_src + s_dst + loop_attr[:, :, None] * c))
            denom = denom + e_l
        w = e / (_bmm(DstOH, denom) + 1e-16)
        wx = _mm(w, R) * _bmm(SrcOH, xp)
        out = _scat(DstOH, wx)
        if add_loops:
            out = out + _mm(e_l / (denom + 1e-16), R) * xp
        return jax.nn.relu(out + b_r[...])

    x = x0_ref[...]
    x = gat(x, W1_ref, As1_ref, Ad1_ref, c1_ref, b1_ref, False)
    x = gat(x, W2_ref, As2_ref, Ad2_ref, c2_ref, b2_ref, True)
    x = gat(x, W3_ref, As3_ref, Ad3_ref, c3_ref, b3_ref, True)

    gemb = jnp.mean(x, axis=1)
    aemb = x[:, :NUM_AGENTS, :]
    gtile = jnp.broadcast_to(gemb[:, None, :], (GB, NUM_AGENTS, HF))
    comb = jnp.concatenate([aemb, gtile], axis=2).reshape(GB * NUM_AGENTS, 2 * HF)
    h = jax.nn.relu(_mm(comb, fc1w_ref[...], jax.lax.Precision.DEFAULT) + fc1b_ref[...])
    pred = _mm(h, fc2w_ref[...], jax.lax.Precision.DEFAULT) + fc2b_ref[...]
    o_ref[...] = pred


@jax.jit
def kernel(tensor, W1, att_src1, att_dst1, W_edge1, att_edge1, b1,
           W2, att_src2, att_dst2, W_edge2, att_edge2, b2,
           W3, att_src3, att_dst3, W_edge3, att_edge3, b3,
           fc1_w, fc1_b, fc2_w, fc2_b):
    Bsz = tensor.shape[0]
    gf = tensor[:, 0, :]
    x0 = gf[:, :NFL].reshape(Bsz, NPG, NNF)
    srcf = gf[:, NFL:NFL + MAX_OBS]
    dstf = gf[:, NFL + MAX_OBS:NFL + 2 * MAX_OBS]
    ea = gf[:, NFL + 2 * MAX_OBS:NFL + 3 * MAX_OBS]

    eye = jnp.eye(HEADS, dtype=jnp.float32)

    def prep(a_srd, a_dst, We, a_e):
        As = (a_srd[0][:, :, None] * eye[:, None, :]).reshape(HF, HEADS)
        Ad = (a_dst[0][:, :, None] * eye[:, None, :]).reshape(HF, HEADS)
        c = (We[0].reshape(HEADS, HID) * a_e[0]).sum(-1).reshape(1, 1, HEADS)
        return As, Ad, c

    As1, Ad1, c1 = prep(att_src1, att_dst1, W_edge1, att_edge1)
    As2, Ad2, c2 = prep(att_src2, att_dst2, W_edge2, att_edge2)
    As3, Ad3, c3 = prep(att_src3, att_dst3, W_edge3, att_edge3)
    R = (eye[:, :, None] * jnp.ones((1, 1, HID))).reshape(HEADS, HF)

    grid = (Bsz // GB,)
    full = lambda shape: pl.BlockSpec(shape, lambda i: tuple(0 for _ in shape))
    out = pl.pallas_call(
        _gat_body,
        grid=grid,
        in_specs=[
            pl.BlockSpec((GB, NPG, NNF), lambda i: (i, 0, 0)),
            pl.BlockSpec((GB, MAX_OBS), lambda i: (i, 0)),
            pl.BlockSpec((GB, MAX_OBS), lambda i: (i, 0)),
            pl.BlockSpec((GB, MAX_OBS), lambda i: (i, 0)),
            full((NNF, HF)), full((HF, HEADS)), full((HF, HEADS)),
            full((1, 1, HEADS)), full((1, 1, HF)),
            full((HF, HF)), full((HF, HEADS)), full((HF, HEADS)),
            full((1, 1, HEADS)), full((1, 1, HF)),
            full((HF, HF)), full((HF, HEADS)), full((HF, HEADS)),
            full((1, 1, HEADS)), full((1, 1, HF)),
            full((HEADS, HF)),
            full((2 * HF, 4 * HID)), full((1, 4 * HID)),
            full((4 * HID, OUT_CH)), full((1, OUT_CH)),
        ],
        out_specs=pl.BlockSpec((GB * NUM_AGENTS, OUT_CH), lambda i: (i, 0)),
        out_shape=jax.ShapeDtypeStruct((Bsz * NUM_AGENTS, OUT_CH), jnp.float32),
    )(x0, srcf, dstf, ea,
      W1, As1, Ad1, c1, b1.reshape(1, 1, HF),
      W2, As2, Ad2, c2, b2.reshape(1, 1, HF),
      W3, As3, Ad3, c3, b3.reshape(1, 1, HF),
      R, fc1_w, fc1_b.reshape(1, 4 * HID), fc2_w, fc2_b.reshape(1, OUT_CH))
    return out.reshape(Bsz, NUM_AGENTS, OUT_CH)


# speedup vs baseline: 71.9896x; 1.0401x over previous
import functools

import jax
import jax.numpy as jnp
from jax.experimental import pallas as pl

B = 1024
NUM_AGENTS = 4
MAX_OBS = 60
NNF = 8
HID = 16
HEADS = 8
OUT_CH = 2
NPG = NUM_AGENTS + MAX_OBS
HF = HEADS * HID
NFL = NPG * NNF

GB = 32


def _bmm(a, b):
    return jax.lax.dot_general(
        a, b, (((2,), (1,)), ((0,), (0,))), preferred_element_type=jnp.float32, precision=jax.lax.Precision.HIGHEST)


def _scat(oh, v):
    return jax.lax.dot_general(
        v, oh, (((1,), (1,)), ((0,), (0,))), preferred_element_type=jnp.float32,
        precision=jax.lax.Precision.HIGHEST).transpose(0, 2, 1)


def _mm(a, b, prec=jax.lax.Precision.HIGHEST):
    return jax.lax.dot_general(
        a, b, (((a.ndim - 1,), (0,)), ((), ())), preferred_element_type=jnp.float32, precision=prec)


def _leaky(a):
    return jnp.where(a > 0, a, 0.2 * a)


def _gat_body(gfn_ref, srcf_ref, dstf_ref, ea_ref,
              W1_ref, As1_ref, Ad1_ref, c1_ref, b1_ref,
              W2_ref, As2_ref, Ad2_ref, c2_ref, b2_ref,
              W3_ref, As3_ref, Ad3_ref, c3_ref, b3_ref,
              R_ref, fc1w_ref, fc1b_ref, fc2w_ref, fc2b_ref,
              o_ref):
    f32 = jnp.float32
    srcf = srcf_ref[...]
    dstf = dstf_ref[...]
    ea = ea_ref[...]
    R = R_ref[...]

    iota = jax.lax.broadcasted_iota(jnp.int32, (GB, MAX_OBS, NPG), 2)
    SrcOH = (srcf.astype(jnp.int32)[:, :, None] == iota).astype(f32)
    DstOH = (dstf.astype(jnp.int32)[:, :, None] == iota).astype(f32)

    ea_sum = jax.lax.dot_general(
        ea, DstOH, (((1,), (1,)), ((0,), (0,))),
        preferred_element_type=f32,
        precision=jax.lax.Precision.HIGHEST)
    cnt = jnp.sum(DstOH, axis=1)
    loop_attr = ea_sum / jnp.maximum(cnt, 1.0)

    def gat(xp, As_r, Ad_r, c_r, b_r, add_loops):
        s_src = _mm(xp, As_r[...])
        s_dst = _mm(xp, Ad_r[...])
        c = c_r[...]
        alpha = _bmm(SrcOH, s_src) + _bmm(DstOH, s_dst) + ea[:, :, None] * c
        e = jnp.exp(_leaky(alpha))
        denom = _scat(DstOH, e)
        if add_loops:
            e_l = jnp.exp(_leaky(s_src + s_dst + loop_attr[:, :, None] * c))
            denom = denom + e_l
        w = e / (_bmm(DstOH, denom) + 1e-16)
        wx = _mm(w, R) * _bmm(SrcOH, xp)
        out = _scat(DstOH, wx)
        if add_loops:
            out = out + _mm(e_l / (denom + 1e-16), R) * xp
        return jax.nn.relu(out + b_r[...])

    xp1 = jax.lax.dot_general(
        gfn_ref[...], W1_ref[...], (((1,), (0,)), ((), ())),
        preferred_element_type=f32, precision=jax.lax.Precision.DEFAULT)
    x = gat(xp1, As1_ref, Ad1_ref, c1_ref, b1_ref, False)
    x = gat(_mm(x, W2_ref[...], jax.lax.Precision.DEFAULT),
            As2_ref, Ad2_ref, c2_ref, b2_ref, True)
    x = gat(_mm(x, W3_ref[...], jax.lax.Precision.DEFAULT),
            As3_ref, Ad3_ref, c3_ref, b3_ref, True)

    gemb = jnp.mean(x, axis=1)
    aemb = x[:, :NUM_AGENTS, :]
    gtile = jnp.broadcast_to(gemb[:, None, :], (GB, NUM_AGENTS, HF))
    comb = jnp.concatenate([aemb, gtile], axis=2).reshape(GB * NUM_AGENTS, 2 * HF)
    h = jax.nn.relu(_mm(comb, fc1w_ref[...], jax.lax.Precision.DEFAULT) + fc1b_ref[...])
    pred = _mm(h, fc2w_ref[...], jax.lax.Precision.DEFAULT) + fc2b_ref[...]
    o_ref[...] = pred


@jax.jit
def kernel(tensor, W1, att_src1, att_dst1, W_edge1, att_edge1, b1,
           W2, att_src2, att_dst2, W_edge2, att_edge2, b2,
           W3, att_src3, att_dst3, W_edge3, att_edge3, b3,
           fc1_w, fc1_b, fc2_w, fc2_b):
    Bsz = tensor.shape[0]
    gf = tensor[:, 0, :]
    gfn = gf[:, :NFL].reshape(Bsz, NPG, NNF).transpose(0, 2, 1)
    srcf = gf[:, NFL:NFL + MAX_OBS]
    dstf = gf[:, NFL + MAX_OBS:NFL + 2 * MAX_OBS]
    ea = gf[:, NFL + 2 * MAX_OBS:NFL + 3 * MAX_OBS]

    eye = jnp.eye(HEADS, dtype=jnp.float32)

    def prep(a_srd, a_dst, We, a_e):
        As = (a_srd[0][:, :, None] * eye[:, None, :]).reshape(HF, HEADS)
        Ad = (a_dst[0][:, :, None] * eye[:, None, :]).reshape(HF, HEADS)
        c = (We[0].reshape(HEADS, HID) * a_e[0]).sum(-1).reshape(1, 1, HEADS)
        return As, Ad, c

    As1, Ad1, c1 = prep(att_src1, att_dst1, W_edge1, att_edge1)
    As2, Ad2, c2 = prep(att_src2, att_dst2, W_edge2, att_edge2)
    As3, Ad3, c3 = prep(att_src3, att_dst3, W_edge3, att_edge3)
    R = (eye[:, :, None] * jnp.ones((1, 1, HID))).reshape(HEADS, HF)

    grid = (Bsz // GB,)
    full = lambda shape: pl.BlockSpec(shape, lambda i: tuple(0 for _ in shape))
    out = pl.pallas_call(
        _gat_body,
        grid=grid,
        in_specs=[
            pl.BlockSpec((GB, NNF, NPG), lambda i: (i, 0, 0)),
            pl.BlockSpec((GB, MAX_OBS), lambda i: (i, 0)),
            pl.BlockSpec((GB, MAX_OBS), lambda i: (i, 0)),
            pl.BlockSpec((GB, MAX_OBS), lambda i: (i, 0)),
            full((NNF, HF)), full((HF, HEADS)), full((HF, HEADS)),
            full((1, 1, HEADS)), full((1, 1, HF)),
            full((HF, HF)), full((HF, HEADS)), full((HF, HEADS)),
            full((1, 1, HEADS)), full((1, 1, HF)),
            full((HF, HF)), full((HF, HEADS)), full((HF, HEADS)),
            full((1, 1, HEADS)), full((1, 1, HF)),
            full((HEADS, HF)),
            full((2 * HF, 4 * HID)), full((1, 4 * HID)),
            full((4 * HID, OUT_CH)), full((1, OUT_CH)),
        ],
        out_specs=pl.BlockSpec((GB * NUM_AGENTS, OUT_CH), lambda i: (i, 0)),
        out_shape=jax.ShapeDtypeStruct((Bsz * NUM_AGENTS, OUT_CH), jnp.float32),
    )(gfn, srcf, dstf, ea,
      W1, As1, Ad1, c1, b1.reshape(1, 1, HF),
      W2, As2, Ad2, c2, b2.reshape(1, 1, HF),
      W3, As3, Ad3, c3, b3.reshape(1, 1, HF),
      R, fc1_w, fc1_b.reshape(1, 4 * HID), fc2_w, fc2_b.reshape(1, OUT_CH))
    return out.reshape(Bsz, NUM_AGENTS, OUT_CH)
